# baseline (speedup 1.0000x reference)
import functools

import jax
import jax.numpy as jnp
from jax import lax
from jax.experimental import pallas as pl
from jax.experimental.pallas import tpu as pltpu
from jax.experimental.pallas import tpu_sc as plsc

T = 2048
D = 768
E = 8
DFF = 3072
EPAD = 128

BT = 128
NB = 40
NROWS = NB * BT

NSC = 16
CH = (2 * T) // NSC
SUB = 64
NSUB = CH // SUB
DTOK = T // 32
VL = 16
_STAGE = 0


def _router_body(x_ref, rw_ref, rb_ref,
                 e1_ref, e2_ref, wa_ref, wb_ref, cur_ref, be_ref, nb_ref):
    xb = x_ref[...]
    logits = jnp.dot(xb, rw_ref[...],
                     preferred_element_type=jnp.float32) + rb_ref[...]
    lane = lax.broadcasted_iota(jnp.int32, (T, EPAD), 1)
    neg = jnp.float32(-1e30)
    logits = jnp.where(lane < E, logits, neg)
    m1 = jnp.max(logits, axis=1, keepdims=True)
    id1 = jnp.min(jnp.where(logits == m1, lane, EPAD), axis=1, keepdims=True)
    l2 = jnp.where(lane == id1, neg, logits)
    m2 = jnp.max(l2, axis=1, keepdims=True)
    id2 = jnp.min(jnp.where(l2 == m2, lane, EPAD), axis=1, keepdims=True)
    wa = 1.0 / (1.0 + jnp.exp(m2 - m1))
    e1_ref[...] = id1
    e2_ref[...] = id2
    wa_ref[...] = wa
    wb_ref[...] = 1.0 - wa

    oh1 = jnp.where(lane == id1, 1.0, 0.0)
    oh2 = jnp.where(lane == id2, 1.0, 0.0)
    srow = lax.broadcasted_iota(jnp.int32, (T, EPAD), 0) // CH
    sel = jnp.where((srow == lane) & (lane < 8), 1.0, 0.0)
    cdims = (((0,), (0,)), ((), ()))
    cc1 = lax.dot_general(sel, oh1, cdims,
                          preferred_element_type=jnp.float32)
    cc2 = lax.dot_general(sel, oh2, cdims,
                          preferred_element_type=jnp.float32)
    chunkcnt = jnp.concatenate([cc1[:8], cc2[:8]], axis=0)

    wrow = lax.broadcasted_iota(jnp.int32, (NSC, NSC), 0)
    wcol = lax.broadcasted_iota(jnp.int32, (NSC, NSC), 1)
    lt16 = jnp.where(wcol < wrow, 1.0, 0.0)
    prior = jnp.dot(lt16, chunkcnt,
                    preferred_element_type=jnp.float32)

    tot = jnp.sum(chunkcnt, axis=0, keepdims=True)
    nbpad = jnp.floor((tot + (BT - 1)) * (1.0 / BT)) * BT
    nbpad = jnp.where(lane[:1, :] < E, nbpad, 0.0)
    li = lax.broadcasted_iota(jnp.int32, (EPAD, EPAD), 0)
    lj = lax.broadcasted_iota(jnp.int32, (EPAD, EPAD), 1)
    ltl = jnp.where(li < lj, 1.0, 0.0)
    offs = jnp.dot(nbpad, ltl,
                   preferred_element_type=jnp.float32)
    cur_ref[...] = (offs + prior).astype(jnp.int32)

    brow = lax.broadcasted_iota(jnp.int32, (NB, EPAD), 0).astype(jnp.float32)
    brow = brow * BT
    lane32 = lax.broadcasted_iota(jnp.int32, (NB, EPAD), 1)
    hit = ((brow >= offs) & (brow < offs + nbpad)
           & (lane32 < E) & (nbpad > 0))
    beb = jnp.sum(jnp.where(hit, lane32.astype(jnp.float32), 0.0),
                  axis=1, keepdims=True)
    anyhit = jnp.sum(jnp.where(hit, 1.0, 0.0), axis=1, keepdims=True)
    be_ref[...] = jnp.where(anyhit > 0, beb, 0.0).astype(jnp.int32)
    nb_ref[...] = (jnp.sum(nbpad, keepdims=True) *
                   (1.0 / BT)).astype(jnp.int32)


def _router(x2d, rw_pad, rb_pad):
    return pl.pallas_call(
        _router_body,
        out_shape=[
            jax.ShapeDtypeStruct((T, 1), jnp.int32),
            jax.ShapeDtypeStruct((T, 1), jnp.int32),
            jax.ShapeDtypeStruct((T, 1), jnp.float32),
            jax.ShapeDtypeStruct((T, 1), jnp.float32),
            jax.ShapeDtypeStruct((NSC, EPAD), jnp.int32),
            jax.ShapeDtypeStruct((NB, 1), jnp.int32),
            jax.ShapeDtypeStruct((1, 1), jnp.int32),
        ],
    )(x2d, rw_pad, rb_pad)


def _dispatch_body(ids_hbm, cur_hbm, x_hbm,
                   xs_hbm, pos_hbm,
                   es_v, posl_v, pos2_v, cur_v, xrow_v, sem):
    cid = lax.axis_index("c")
    w = lax.axis_index("s")
    btok = (w % 8) * CH

    pltpu.sync_copy(ids_hbm.at[pl.ds(w * CH, CH)], es_v)
    pltpu.sync_copy(cur_hbm.at[pl.ds(w * EPAD, VL)], cur_v)
    curvec = cur_v[...]
    cursor = [curvec[e] for e in range(E)]
    zero = jnp.zeros((VL,), jnp.int32)

    for j in range(CH // VL):
        v = es_v[pl.ds(j * VL, VL)]
        pv = zero
        for e in range(E):
            m = v == e
            mi = jnp.where(m, 1, 0).astype(jnp.int32)
            cum = jnp.cumsum(mi)
            pv = jnp.where(m, cursor[e] + cum - 1, pv)
            cursor[e] = cursor[e] + cum[VL - 1]
        pv = jnp.clip(pv, 0, NROWS - 1)
        posl_v[pl.ds(j * VL, VL)] = pv
        pos2_v[(j * VL) // SUB, pl.ds((j * VL) % SUB, VL)] = pv

    @pl.when(cid == 0)
    def _():
        pltpu.sync_copy(posl_v, pos_hbm.at[pl.ds(w * CH, CH)])

    for j in range(NSUB):
        @pl.when(cid == j % 2)
        def _():
            pltpu.sync_copy(x_hbm.at[pl.ds(btok + j * SUB, SUB)], xrow_v)
            pltpu.async_copy(xrow_v, xs_hbm.at[pos2_v.at[j]], sem).wait()


def _dispatch(ids_flat, cur_flat, x2d):
    mesh = plsc.VectorSubcoreMesh(core_axis_name="c", subcore_axis_name="s")
    f = functools.partial(
        pl.kernel, mesh=mesh,
        out_type=[
            jax.ShapeDtypeStruct((NROWS, D), jnp.float32),
            jax.ShapeDtypeStruct((2 * T,), jnp.int32),
        ],
        scratch_types=[
            pltpu.VMEM((CH,), jnp.int32),
            pltpu.VMEM((CH,), jnp.int32),
            pltpu.VMEM((NSUB, SUB), jnp.int32),
            pltpu.VMEM((VL,), jnp.int32),
            pltpu.VMEM((SUB, D), jnp.float32),
            pltpu.SemaphoreType.DMA,
        ],
        compiler_params=pltpu.CompilerParams(needs_layout_passes=False),
    )(_dispatch_body)
    return f(ids_flat, cur_flat, x2d)


def _ffn_outer(be_ref, nb_ref, xs_hbm, w1_hbm, b1_hbm, w2_hbm, b2_hbm,
               out_hbm):
    nb = jnp.clip(nb_ref[0], 1, NB)

    def inner(xs_blk, w1_blk, b1_blk, w2_blk, b2_blk, out_blk):
        h = jnp.dot(xs_blk[...], w1_blk[0],
                    preferred_element_type=jnp.float32) + b1_blk[0]
        h = jnp.maximum(h, 0.0)
        out_blk[...] = jnp.dot(h, w2_blk[0],
                               preferred_element_type=jnp.float32) + b2_blk[0]

    def _we(b):
        return (jnp.clip(be_ref[b], 0, E - 1), 0, 0)

    la = pl.Buffered(buffer_count=2, use_lookahead=True)
    pltpu.emit_pipeline(
        inner,
        grid=(nb,),
        in_specs=[
            pl.BlockSpec((BT, D), lambda b: (b, 0)),
            pl.BlockSpec((1, D, DFF), _we, pipeline_mode=la),
            pl.BlockSpec((1, 1, DFF), _we),
            pl.BlockSpec((1, DFF, D), _we, pipeline_mode=la),
            pl.BlockSpec((1, 1, D), _we),
        ],
        out_specs=[pl.BlockSpec((BT, D), lambda b: (b, 0))],
    )(xs_hbm, w1_hbm, b1_hbm, w2_hbm, b2_hbm, out_hbm)


def _ffn(be, nb, xs, w1, b1r, w2, b2r):
    return pl.pallas_call(
        _ffn_outer,
        in_specs=[
            pl.BlockSpec(memory_space=pltpu.SMEM),
            pl.BlockSpec(memory_space=pltpu.SMEM),
            pl.BlockSpec(memory_space=pl.ANY),
            pl.BlockSpec(memory_space=pl.ANY),
            pl.BlockSpec(memory_space=pl.ANY),
            pl.BlockSpec(memory_space=pl.ANY),
            pl.BlockSpec(memory_space=pl.ANY),
        ],
        out_specs=pl.BlockSpec(memory_space=pl.ANY),
        out_shape=jax.ShapeDtypeStruct((NROWS, D), jnp.float32),
    )(be, nb, xs, w1, b1r, w2, b2r)


def _combine_body(eo_hbm, pos_hbm, wts_hbm, out_hbm,
                  pa_v, pb_v, wa_v, wb_v, ra_v, rb_v, sema, semb):
    cid = lax.axis_index("c")
    sid = lax.axis_index("s")
    wid = sid * 2 + cid
    t0 = wid * DTOK
    pltpu.sync_copy(pos_hbm.at[pl.ds(t0, DTOK)], pa_v)
    pltpu.sync_copy(pos_hbm.at[pl.ds(T + t0, DTOK)], pb_v)
    pltpu.sync_copy(wts_hbm.at[pl.ds(t0, DTOK)], wa_v.at[pl.ds(0, DTOK)])
    pltpu.sync_copy(wts_hbm.at[pl.ds(T + t0, DTOK)], wb_v.at[pl.ds(0, DTOK)])
    for k in range(DTOK // VL):
        sl = pl.ds(k * VL, VL)
        pa_v[sl] = jnp.clip(pa_v[sl], 0, NROWS - 1)
        pb_v[sl] = jnp.clip(pb_v[sl], 0, NROWS - 1)
    cpa = pltpu.async_copy(eo_hbm.at[pa_v], ra_v, sema)
    cpb = pltpu.async_copy(eo_hbm.at[pb_v], rb_v, semb)
    cpa.wait()
    cpb.wait()

    def tok_body(t, carry):
        a = wa_v[pl.ds(t, VL)][0]
        bw = wb_v[pl.ds(t, VL)][0]
        for j in range(D // VL):
            sl = pl.ds(j * VL, VL)
            ra_v[t, sl] = a * ra_v[t, sl] + bw * rb_v[t, sl]
        return carry

    lax.fori_loop(0, DTOK, tok_body, 0)
    pltpu.sync_copy(ra_v, out_hbm.at[pl.ds(t0, DTOK)])


def _combine(eo, pos_flat, wts_flat):
    mesh = plsc.VectorSubcoreMesh(core_axis_name="c", subcore_axis_name="s")
    f = functools.partial(
        pl.kernel, mesh=mesh,
        out_type=jax.ShapeDtypeStruct((T, D), jnp.float32),
        scratch_types=[
            pltpu.VMEM((DTOK,), jnp.int32),
            pltpu.VMEM((DTOK,), jnp.int32),
            pltpu.VMEM((DTOK + VL,), jnp.float32),
            pltpu.VMEM((DTOK + VL,), jnp.float32),
            pltpu.VMEM((DTOK, D), jnp.float32),
            pltpu.VMEM((DTOK, D), jnp.float32),
            pltpu.SemaphoreType.DMA,
            pltpu.SemaphoreType.DMA,
        ],
        compiler_params=pltpu.CompilerParams(needs_layout_passes=False),
    )(_combine_body)
    return f(eo, pos_flat, wts_flat)


@jax.jit
def _moe(x2d, rw_pad, rb_pad, w1, b1r, w2, b2r):
    e1, e2, wa, wb, cur, be, nb = _router(x2d, rw_pad, rb_pad)
    if _STAGE == 1:
        return (e1, e2, wa, wb, cur, be)
    ids_flat = jnp.concatenate([e1.reshape(T), e2.reshape(T)])
    wts_flat = jnp.concatenate([wa.reshape(T), wb.reshape(T)])
    xs, pos_flat = _dispatch(ids_flat, cur.reshape(NSC * EPAD), x2d)
    if _STAGE == 2:
        return (xs, pos_flat)
    eo = _ffn(be.reshape(NB), nb.reshape(1), xs, w1, b1r, w2, b2r)
    if _STAGE == 3:
        return (eo, pos_flat)
    return _combine(eo, pos_flat, wts_flat)


def kernel(x, router_w, router_b, w1, b1, w2, b2):
    b_, l_, d_ = x.shape
    x2d = x.reshape(l_, d_)
    rw_pad = jnp.zeros((D, EPAD), jnp.float32).at[:, :E].set(router_w)
    rb_pad = jnp.zeros((1, EPAD), jnp.float32).at[0, :E].set(router_b)
    out = _moe(x2d, rw_pad, rb_pad, w1,
               b1.reshape(E, 1, DFF), w2, b2.reshape(E, 1, D))
    if _STAGE:
        return out
    return out.reshape(b_, l_, d_)


# speedup vs baseline: 1.1710x; 1.1710x over previous
import functools

import jax
import jax.numpy as jnp
from jax import lax
from jax.experimental import pallas as pl
from jax.experimental.pallas import tpu as pltpu
from jax.experimental.pallas import tpu_sc as plsc

T = 2048
D = 768
E = 8
DFF = 3072
EPAD = 128

BT = 256
NB = 24
NROWS = NB * BT

NSC = 16
CH = (2 * T) // NSC
SUB = 64
NSUB = CH // SUB
DTOK = T // 32
VL = 16


def _router_body(x_ref, rw_ref, rb_ref,
                 ids_ref, wts_ref, cur_ref, meta_ref):
    xb = x_ref[...]
    logits = jnp.dot(xb, rw_ref[...],
                     preferred_element_type=jnp.float32) + rb_ref[...]
    lane = lax.broadcasted_iota(jnp.int32, (T, EPAD), 1)
    neg = jnp.float32(-1e30)
    logits = jnp.where(lane < E, logits, neg)
    m1 = jnp.max(logits, axis=1, keepdims=True)
    id1 = jnp.min(jnp.where(logits == m1, lane, EPAD), axis=1, keepdims=True)
    l2 = jnp.where(lane == id1, neg, logits)
    m2 = jnp.max(l2, axis=1, keepdims=True)
    id2 = jnp.min(jnp.where(l2 == m2, lane, EPAD), axis=1, keepdims=True)
    wa = 1.0 / (1.0 + jnp.exp(m2 - m1))
    wb = 1.0 - wa

    tok = lax.broadcasted_iota(jnp.int32, (T, EPAD), 0)
    vmask = jnp.where((tok % EPAD) == lane, 1.0, 0.0)
    umask = jnp.where((tok // EPAD) == lane, 1.0, 0.0)
    cdims0 = (((0,), (0,)), ((), ()))
    hp = lax.Precision.HIGHEST

    def lanemajor(col, prec):
        w = col * vmask
        return lax.dot_general(umask, w, cdims0, precision=prec,
                               preferred_element_type=jnp.float32)[:VL]

    ids_ref[...] = jnp.concatenate(
        [lanemajor(id1.astype(jnp.float32), None),
         lanemajor(id2.astype(jnp.float32), None)], axis=0).astype(jnp.int32)
    wts_ref[...] = jnp.concatenate(
        [lanemajor(wa, hp), lanemajor(wb, hp)], axis=0)

    oh1 = jnp.where(lane == id1, 1.0, 0.0)
    oh2 = jnp.where(lane == id2, 1.0, 0.0)
    srow = lax.broadcasted_iota(jnp.int32, (T, EPAD), 0) // CH
    sel = jnp.where((srow == lane) & (lane < 8), 1.0, 0.0)
    cdims = (((0,), (0,)), ((), ()))
    cc1 = lax.dot_general(sel, oh1, cdims,
                          preferred_element_type=jnp.float32)
    cc2 = lax.dot_general(sel, oh2, cdims,
                          preferred_element_type=jnp.float32)
    chunkcnt = jnp.concatenate([cc1[:8], cc2[:8]], axis=0)

    wrow = lax.broadcasted_iota(jnp.int32, (NSC, NSC), 0)
    wcol = lax.broadcasted_iota(jnp.int32, (NSC, NSC), 1)
    lt16 = jnp.where(wcol < wrow, 1.0, 0.0)
    prior = jnp.dot(lt16, chunkcnt,
                    preferred_element_type=jnp.float32)

    tot = jnp.sum(chunkcnt, axis=0, keepdims=True)
    nbpad = jnp.floor((tot + (BT - 1)) * (1.0 / BT)) * BT
    nbpad = jnp.where(lane[:1, :] < E, nbpad, 0.0)
    li = lax.broadcasted_iota(jnp.int32, (EPAD, EPAD), 0)
    lj = lax.broadcasted_iota(jnp.int32, (EPAD, EPAD), 1)
    ltl = jnp.where(li < lj, 1.0, 0.0)
    offs = jnp.dot(nbpad, ltl,
                   preferred_element_type=jnp.float32)
    cur_ref[...] = (offs + prior).astype(jnp.int32)

    brow = lax.broadcasted_iota(jnp.int32, (NB, EPAD), 0).astype(jnp.float32)
    brow = brow * BT
    lane32 = lax.broadcasted_iota(jnp.int32, (NB, EPAD), 1)
    hit = ((brow >= offs) & (brow < offs + nbpad)
           & (lane32 < E) & (nbpad > 0))
    beb = jnp.sum(jnp.where(hit, lane32.astype(jnp.float32), 0.0),
                  axis=1, keepdims=True)
    anyhit = jnp.sum(jnp.where(hit, 1.0, 0.0), axis=1, keepdims=True)
    becol = jnp.where(anyhit > 0, beb, 0.0)
    bsel = lax.broadcasted_iota(jnp.int32, (NB, EPAD), 0)
    blane = lax.broadcasted_iota(jnp.int32, (NB, EPAD), 1)
    bv = jnp.where(bsel == blane, 1.0, 0.0) * becol
    berow = jnp.sum(bv, axis=0, keepdims=True)
    nbval = jnp.sum(nbpad, keepdims=True) * (1.0 / BT)
    meta_ref[...] = (berow + jnp.where(lane[:1, :] == EPAD - 1,
                                       nbval, 0.0)).astype(jnp.int32)


def _router(x2d, rw_pad, rb_pad):
    return pl.pallas_call(
        _router_body,
        out_shape=[
            jax.ShapeDtypeStruct((2 * VL, EPAD), jnp.int32),
            jax.ShapeDtypeStruct((2 * VL, EPAD), jnp.float32),
            jax.ShapeDtypeStruct((NSC, EPAD), jnp.int32),
            jax.ShapeDtypeStruct((1, EPAD), jnp.int32),
        ],
    )(x2d, rw_pad, rb_pad)


def _dispatch_body(ids_hbm, cur_hbm, x_hbm,
                   xs_hbm, pos_hbm,
                   es_v, posl_v, pos2_v, cur_v, xrow_v, sem):
    cid = lax.axis_index("c")
    w = lax.axis_index("s")
    btok = (w % 8) * CH

    pltpu.sync_copy(ids_hbm.at[pl.ds(w * CH, CH)], es_v)
    pltpu.sync_copy(cur_hbm.at[pl.ds(w * EPAD, VL)], cur_v)
    curvec = cur_v[...]
    cursor = [curvec[e] for e in range(E)]
    zero = jnp.zeros((VL,), jnp.int32)

    for j in range(CH // VL):
        v = es_v[pl.ds(j * VL, VL)]
        pv = zero
        for e in range(E):
            m = v == e
            mi = jnp.where(m, 1, 0).astype(jnp.int32)
            cum = jnp.cumsum(mi)
            pv = jnp.where(m, cursor[e] + cum - 1, pv)
            cursor[e] = cursor[e] + cum[VL - 1]
        pv = jnp.clip(pv, 0, NROWS - 1)
        posl_v[pl.ds(j * VL, VL)] = pv
        pos2_v[(j * VL) // SUB, pl.ds((j * VL) % SUB, VL)] = pv

    @pl.when(cid == 0)
    def _():
        pltpu.sync_copy(posl_v, pos_hbm.at[pl.ds(w * CH, CH)])

    for j in range(NSUB):
        @pl.when(cid == j % 2)
        def _():
            pltpu.sync_copy(x_hbm.at[pl.ds(btok + j * SUB, SUB)], xrow_v)
            pltpu.async_copy(xrow_v, xs_hbm.at[pos2_v.at[j]], sem).wait()


def _dispatch(ids_flat, cur_flat, x2d):
    mesh = plsc.VectorSubcoreMesh(core_axis_name="c", subcore_axis_name="s")
    f = functools.partial(
        pl.kernel, mesh=mesh,
        out_type=[
            jax.ShapeDtypeStruct((NROWS, D), jnp.float32),
            jax.ShapeDtypeStruct((2 * T,), jnp.int32),
        ],
        scratch_types=[
            pltpu.VMEM((CH,), jnp.int32),
            pltpu.VMEM((CH,), jnp.int32),
            pltpu.VMEM((NSUB, SUB), jnp.int32),
            pltpu.VMEM((VL,), jnp.int32),
            pltpu.VMEM((SUB, D), jnp.float32),
            pltpu.SemaphoreType.DMA,
        ],
        compiler_params=pltpu.CompilerParams(needs_layout_passes=False),
    )(_dispatch_body)
    return f(ids_flat, cur_flat, x2d)


def _ffn_outer(meta_ref, xs_hbm, w1_hbm, b1_hbm, w2_hbm, b2_hbm,
               out_hbm):
    nb = jnp.clip(meta_ref[EPAD - 1], 1, NB)

    def inner(xs_blk, w1_blk, b1_blk, w2_blk, b2_blk, out_blk):
        h = jnp.dot(xs_blk[...], w1_blk[0],
                    preferred_element_type=jnp.float32) + b1_blk[0]
        h = jnp.maximum(h, 0.0)
        out_blk[...] = jnp.dot(h, w2_blk[0],
                               preferred_element_type=jnp.float32) + b2_blk[0]

    def _we(b):
        return (jnp.clip(meta_ref[b], 0, E - 1), 0, 0)

    la = pl.Buffered(buffer_count=2, use_lookahead=True)
    pltpu.emit_pipeline(
        inner,
        grid=(nb,),
        in_specs=[
            pl.BlockSpec((BT, D), lambda b: (b, 0)),
            pl.BlockSpec((1, D, DFF), _we, pipeline_mode=la),
            pl.BlockSpec((1, 1, DFF), _we),
            pl.BlockSpec((1, DFF, D), _we, pipeline_mode=la),
            pl.BlockSpec((1, 1, D), _we),
        ],
        out_specs=[pl.BlockSpec((BT, D), lambda b: (b, 0))],
    )(xs_hbm, w1_hbm, b1_hbm, w2_hbm, b2_hbm, out_hbm)


def _ffn(meta, xs, w1, b1r, w2, b2r):
    return pl.pallas_call(
        _ffn_outer,
        in_specs=[
            pl.BlockSpec(memory_space=pltpu.SMEM),
            pl.BlockSpec(memory_space=pl.ANY),
            pl.BlockSpec(memory_space=pl.ANY),
            pl.BlockSpec(memory_space=pl.ANY),
            pl.BlockSpec(memory_space=pl.ANY),
            pl.BlockSpec(memory_space=pl.ANY),
        ],
        out_specs=pl.BlockSpec(memory_space=pl.ANY),
        out_shape=jax.ShapeDtypeStruct((NROWS, D), jnp.float32),
    )(meta, xs, w1, b1r, w2, b2r)


def _combine_body(eo_hbm, pos_hbm, wts_hbm, out_hbm,
                  pa_v, pb_v, wa_v, wb_v, ra_v, rb_v, sema, semb):
    cid = lax.axis_index("c")
    sid = lax.axis_index("s")
    wid = sid * 2 + cid
    t0 = wid * DTOK
    pltpu.sync_copy(pos_hbm.at[pl.ds(t0, DTOK)], pa_v)
    pltpu.sync_copy(pos_hbm.at[pl.ds(T + t0, DTOK)], pb_v)
    pltpu.sync_copy(wts_hbm.at[pl.ds(t0, DTOK)], wa_v.at[pl.ds(0, DTOK)])
    pltpu.sync_copy(wts_hbm.at[pl.ds(T + t0, DTOK)], wb_v.at[pl.ds(0, DTOK)])
    for k in range(DTOK // VL):
        sl = pl.ds(k * VL, VL)
        pa_v[sl] = jnp.clip(pa_v[sl], 0, NROWS - 1)
        pb_v[sl] = jnp.clip(pb_v[sl], 0, NROWS - 1)
    cpa = pltpu.async_copy(eo_hbm.at[pa_v], ra_v, sema)
    cpb = pltpu.async_copy(eo_hbm.at[pb_v], rb_v, semb)
    cpa.wait()
    cpb.wait()

    def tok_body(t, carry):
        a = wa_v[pl.ds(t, VL)][0]
        bw = wb_v[pl.ds(t, VL)][0]
        for j in range(D // VL):
            sl = pl.ds(j * VL, VL)
            ra_v[t, sl] = a * ra_v[t, sl] + bw * rb_v[t, sl]
        return carry

    lax.fori_loop(0, DTOK, tok_body, 0)
    pltpu.sync_copy(ra_v, out_hbm.at[pl.ds(t0, DTOK)])


def _combine(eo, pos_flat, wts_flat):
    mesh = plsc.VectorSubcoreMesh(core_axis_name="c", subcore_axis_name="s")
    f = functools.partial(
        pl.kernel, mesh=mesh,
        out_type=jax.ShapeDtypeStruct((T, D), jnp.float32),
        scratch_types=[
            pltpu.VMEM((DTOK,), jnp.int32),
            pltpu.VMEM((DTOK,), jnp.int32),
            pltpu.VMEM((DTOK + VL,), jnp.float32),
            pltpu.VMEM((DTOK + VL,), jnp.float32),
            pltpu.VMEM((DTOK, D), jnp.float32),
            pltpu.VMEM((DTOK, D), jnp.float32),
            pltpu.SemaphoreType.DMA,
            pltpu.SemaphoreType.DMA,
        ],
        compiler_params=pltpu.CompilerParams(needs_layout_passes=False),
    )(_combine_body)
    return f(eo, pos_flat, wts_flat)


@jax.jit
def _moe(x2d, rw_pad, rb_pad, w1, b1r, w2, b2r):
    ids_lm, wts_lm, cur, meta = _router(x2d, rw_pad, rb_pad)
    ids_flat = ids_lm.reshape(2 * T)
    wts_flat = wts_lm.reshape(2 * T)
    xs, pos_flat = _dispatch(ids_flat, cur.reshape(NSC * EPAD), x2d)
    eo = _ffn(meta.reshape(EPAD), xs, w1, b1r, w2, b2r)
    return _combine(eo, pos_flat, wts_flat)


def kernel(x, router_w, router_b, w1, b1, w2, b2):
    b_, l_, d_ = x.shape
    x2d = x.reshape(l_, d_)
    rw_pad = jnp.zeros((D, EPAD), jnp.float32).at[:, :E].set(router_w)
    rb_pad = jnp.zeros((1, EPAD), jnp.float32).at[0, :E].set(router_b)
    out = _moe(x2d, rw_pad, rb_pad, w1,
               b1.reshape(E, 1, DFF), w2, b2.reshape(E, 1, D))
    return out.reshape(b_, l_, d_)
